# baseline (speedup 1.0000x reference)
import functools

import jax
import jax.numpy as jnp
from jax import lax
from jax.experimental import pallas as pl
from jax.experimental.pallas import tpu as pltpu
from jax.experimental.pallas import tpu_sc as plsc

B = 16384
D = 32
N_ROWS = 1000000
NFULL = N_ROWS // 128
TAIL = N_ROWS - NFULL * 128
PADW = 128
NPROWS = (NFULL + 1) * 32

_info = plsc.get_sparse_core_info()
NC, NS, L = _info.num_cores, _info.num_subcores, _info.num_lanes
NW = NC * NS
BPW = B // NW
CH = 128
NCHK = BPW // CH

NSLOT = 4
NGRP = -(-245 // NSLOT)


def _transpose_pack(tb, ob, nrows):
    iota = lax.iota(jnp.int32, L)
    for q in range(nrows // L):
        rows = q * L + iota
        prow = rows >> 2
        pcol0 = (rows & 3) * D
        for j in range(D):
            v = tb[j, pl.ds(q * L, L)]
            plsc.store_scatter(ob, [prow, pcol0 + j], v)


def _detile_kernel(ut_hbm, it_hbm, ut_tail, it_tail, u128_hbm, i128_hbm,
                   tbs0, tbs1, tbs2, tbs3, obs0, obs1, obs2, obs3,
                   si0, si1, si2, si3, so0, so1, so2, so3):
    wid = lax.axis_index("s") * NC + lax.axis_index("c")
    lo_col = 244 * wid + jnp.minimum(wid, 4)
    ncols = 244 + (wid < 4).astype(jnp.int32)
    tbs = [tbs0, tbs1, tbs2, tbs3]
    obs = [obs0, obs1, obs2, obs3]
    sin = [si0, si1, si2, si3]
    sout = [so0, so1, so2, so3]

    def do_table(src, dst):
        def start_in(b, s):
            c0 = pl.multiple_of((lo_col + b) * 128, 128)
            pltpu.make_async_copy(
                src.at[pl.ds(0, D), pl.ds(c0, 128)], tbs[s], sin[s]).start()

        def out_copy(b, s):
            r0 = pl.multiple_of((lo_col + b) * 32, 32)
            return pltpu.make_async_copy(
                obs[s], dst.at[pl.ds(r0, 32), pl.ds(0, PADW)], sout[s])

        for s in range(NSLOT):
            start_in(s, s)

        def group(g, carry):
            for s in range(NSLOT):
                b = g * NSLOT + s

                @pl.when(b < ncols)
                def _process(b=b, s=s):
                    c0 = pl.multiple_of((lo_col + b) * 128, 128)
                    pltpu.make_async_copy(
                        src.at[pl.ds(0, D), pl.ds(c0, 128)], tbs[s],
                        sin[s]).wait()

                    @pl.when(g > 0)
                    def _wait_prev_out():
                        out_copy(b, s).wait()

                    _transpose_pack(tbs[s], obs[s], 128)
                    out_copy(b, s).start()

                    @pl.when(b + NSLOT < ncols)
                    def _prefetch():
                        start_in(b + NSLOT, s)
            return carry

        lax.fori_loop(0, NGRP, group, 0)
        for s in range(NSLOT):
            out_copy(0, s).wait()

    do_table(ut_hbm, u128_hbm)
    do_table(it_hbm, i128_hbm)

    @pl.when(wid == NW - 1)
    def _tail():
        for tail_src, dst in ((ut_tail, u128_hbm), (it_tail, i128_hbm)):
            pltpu.sync_copy(tail_src, tbs0)
            _transpose_pack(tbs0, obs0, TAIL)
            pltpu.sync_copy(
                obs0.at[pl.ds(0, TAIL // 4), pl.ds(0, PADW)],
                dst.at[pl.ds(NFULL * 32, TAIL // 4), pl.ds(0, PADW)])


_detile = functools.partial(
    pl.kernel,
    mesh=plsc.VectorSubcoreMesh(core_axis_name="c", subcore_axis_name="s"),
    out_type=(jax.ShapeDtypeStruct((NPROWS, PADW), jnp.float32),
              jax.ShapeDtypeStruct((NPROWS, PADW), jnp.float32)),
    scratch_types=(
        [pltpu.VMEM((D, 128), jnp.float32)] * NSLOT
        + [pltpu.VMEM((D, PADW), jnp.float32)] * NSLOT
        + [pltpu.SemaphoreType.DMA] * (2 * NSLOT)
    ),
    compiler_params=pltpu.CompilerParams(needs_layout_passes=False),
)(_detile_kernel)


def _scores_kernel(users_hbm, pos_hbm, neg_hbm, u128_hbm, i128_hbm,
                   out_hbm, idx_u, idx_p, idx_n, pr_u, pr_p, pr_n,
                   du0, dp0, dn0, du1, dp1, dn1, scores_v, sem0, sem1):
    wid = lax.axis_index("s") * NC + lax.axis_index("c")
    base = wid * BPW

    pltpu.sync_copy(users_hbm.at[pl.ds(base, BPW)], idx_u)
    pltpu.sync_copy(pos_hbm.at[pl.ds(base, BPW)], idx_p)
    pltpu.sync_copy(neg_hbm.at[pl.ds(base, BPW)], idx_n)

    def split(idx, pr):
        def body(g, carry):
            v = idx[pl.ds(g * L, L)]
            pr[pl.ds(g * L, L)] = v >> 2
            idx[pl.ds(g * L, L)] = (v & 3) * D
            return carry
        lax.fori_loop(0, BPW // L, body, 0)

    split(idx_u, pr_u)
    split(idx_p, pr_p)
    split(idx_n, pr_n)

    slots = [(du0, dp0, dn0, sem0), (du1, dp1, dn1, sem1)]

    def copies(c):
        du, dp, dn, sem = slots[c % 2]
        r = pl.ds(c * CH, CH)
        return (
            pltpu.make_async_copy(u128_hbm.at[pr_u.at[r]], du, sem),
            pltpu.make_async_copy(i128_hbm.at[pr_p.at[r]], dp, sem),
            pltpu.make_async_copy(i128_hbm.at[pr_n.at[r]], dn, sem),
        )

    for cp in copies(0):
        cp.start()
    for cp in copies(1):
        cp.start()

    iota = lax.iota(jnp.int32, L)
    for c in range(NCHK):
        for cp in copies(c):
            cp.wait()
        du, dp, dn, _ = slots[c % 2]

        def grp(g, carry, du=du, dp=dp, dn=dn, c=c):
            rows = g * L + iota
            o = pl.ds(c * CH + g * L, L)
            cu = idx_u[o]
            cp_ = idx_p[o]
            cn = idx_n[o]
            acc = jnp.zeros((L,), jnp.float32)
            for j in range(D):
                u = plsc.load_gather(du, [rows, cu + j])
                p = plsc.load_gather(dp, [rows, cp_ + j])
                n = plsc.load_gather(dn, [rows, cn + j])
                acc = acc + u * (n - p)
            scores_v[o] = acc
            return carry

        lax.fori_loop(0, CH // L, grp, 0)
        if c + 2 < NCHK:
            for cp in copies(c + 2):
                cp.start()

    pltpu.sync_copy(scores_v, out_hbm.at[pl.ds(base, BPW)])


_sc_scores = functools.partial(
    pl.kernel,
    mesh=plsc.VectorSubcoreMesh(core_axis_name="c", subcore_axis_name="s"),
    out_type=jax.ShapeDtypeStruct((B,), jnp.float32),
    scratch_types=(
        [pltpu.VMEM((BPW,), jnp.int32)] * 6
        + [pltpu.VMEM((CH, PADW), jnp.float32)] * 6
        + [pltpu.VMEM((BPW,), jnp.float32)]
        + [pltpu.SemaphoreType.DMA] * 2
    ),
    compiler_params=pltpu.CompilerParams(needs_layout_passes=False),
)(_scores_kernel)


def _softplus_mean_kernel(x_ref, o_ref):
    x = x_ref[...]
    o_ref[...] = (jnp.sum(jnp.log(1.0 + jnp.exp(x))) * (1.0 / B))[None, None]


def kernel(users, positive_items, negative_items, user_embedding, item_embedding):
    ut = user_embedding.T
    it = item_embedding.T
    ut_tail = jnp.pad(ut[:, NFULL * 128:], ((0, 0), (0, 128 - TAIL)))
    it_tail = jnp.pad(it[:, NFULL * 128:], ((0, 0), (0, 128 - TAIL)))
    u128, i128 = _detile(ut, it, ut_tail, it_tail)
    scores = _sc_scores(users, positive_items, negative_items, u128, i128)
    loss = pl.pallas_call(
        _softplus_mean_kernel,
        out_shape=jax.ShapeDtypeStruct((1, 1), jnp.float32),
    )(scores.reshape(128, 128))
    return loss.reshape(())


# speedup vs baseline: 2.6168x; 2.6168x over previous
import functools

import jax
import jax.numpy as jnp
from jax import lax
from jax.experimental import pallas as pl
from jax.experimental.pallas import tpu as pltpu
from jax.experimental.pallas import tpu_sc as plsc

B = 16384
D = 32
N_ROWS = 1000000
NFULL = N_ROWS // 128
TAIL = N_ROWS - NFULL * 128
PADW = 128
NPROWS = (NFULL + 1) * 32

_info = plsc.get_sparse_core_info()
NC, NS, L = _info.num_cores, _info.num_subcores, _info.num_lanes
NW = NC * NS
BPW = B // NW
CH = 128
NCHK = BPW // CH

W = 256
OBP = 132
NSLOT = 4
NBLK_MAX = -(-245 // 2)


def _transpose_pack(tb, ob, ncols_blk):
    iota = lax.iota(jnp.int32, L)

    def q_body(q, carry):
        i = q * L + iota
        prow = i >> 2
        m = i & 3
        for j in range(D):
            v = tb[j, pl.ds(q * L, L)]
            plsc.store_scatter(ob, [prow, m + 4 * j], v)
        return carry

    lax.fori_loop(0, ncols_blk // L, q_body, 0)


def _detile_kernel(ut_hbm, it_hbm, ut_tail, it_tail, u128_hbm, i128_hbm,
                   tbs0, tbs1, tbs2, tbs3, obs0, obs1, obs2, obs3,
                   si0, si1, si2, si3, so0, so1, so2, so3):
    wid = lax.axis_index("s") * NC + lax.axis_index("c")
    lo_col = 244 * wid + jnp.minimum(wid, 4)
    ncols = 244 + (wid < 4).astype(jnp.int32)
    nblk = (ncols + 1) // 2
    tbs = [tbs0, tbs1, tbs2, tbs3]
    obs = [obs0, obs1, obs2, obs3]
    sin = [si0, si1, si2, si3]
    sout = [so0, so1, so2, so3]

    def do_table(src, dst):
        def start_in(b, s):
            c0 = pl.multiple_of((lo_col + 2 * b) * 128, 128)
            pltpu.make_async_copy(
                src.at[pl.ds(0, D), pl.ds(c0, W)], tbs[s], sin[s]).start()

        def out_copy(b, s):
            r0 = pl.multiple_of((lo_col + 2 * b) * 32, 32)
            return pltpu.make_async_copy(
                obs[s].at[pl.ds(0, W // 4), pl.ds(0, PADW)],
                dst.at[pl.ds(r0, W // 4), pl.ds(0, PADW)], sout[s])

        for s in range(NSLOT):
            start_in(s, s)

        def group(g, carry):
            for s in range(NSLOT):
                b = g * NSLOT + s

                @pl.when(b < nblk)
                def _process(b=b, s=s):
                    c0 = pl.multiple_of((lo_col + 2 * b) * 128, 128)
                    pltpu.make_async_copy(
                        src.at[pl.ds(0, D), pl.ds(c0, W)], tbs[s],
                        sin[s]).wait()

                    @pl.when(g > 0)
                    def _wait_prev_out():
                        out_copy(b, s).wait()

                    _transpose_pack(tbs[s], obs[s], W)
                    out_copy(b, s).start()

                    @pl.when(b + NSLOT < nblk)
                    def _prefetch():
                        start_in(b + NSLOT, s)
            return carry

        lax.fori_loop(0, -(-NBLK_MAX // NSLOT), group, 0)
        for s in range(NSLOT):
            out_copy(0, s).wait()

    do_table(ut_hbm, u128_hbm)
    do_table(it_hbm, i128_hbm)

    @pl.when(wid == NW - 1)
    def _tail():
        for tail_src, dst in ((ut_tail, u128_hbm), (it_tail, i128_hbm)):
            pltpu.sync_copy(tail_src, tbs0.at[pl.ds(0, D), pl.ds(0, 128)])
            _transpose_pack(tbs0, obs0, 128)
            pltpu.sync_copy(
                obs0.at[pl.ds(0, 32), pl.ds(0, PADW)],
                dst.at[pl.ds(NFULL * 32, 32), pl.ds(0, PADW)])


_detile = functools.partial(
    pl.kernel,
    mesh=plsc.VectorSubcoreMesh(core_axis_name="c", subcore_axis_name="s"),
    out_type=(jax.ShapeDtypeStruct((NPROWS, PADW), jnp.float32),
              jax.ShapeDtypeStruct((NPROWS, PADW), jnp.float32)),
    scratch_types=(
        [pltpu.VMEM((D, W), jnp.float32)] * NSLOT
        + [pltpu.VMEM((W // 4, OBP), jnp.float32)] * NSLOT
        + [pltpu.SemaphoreType.DMA] * (2 * NSLOT)
    ),
    compiler_params=pltpu.CompilerParams(needs_layout_passes=False),
)(_detile_kernel)


def _scores_kernel(users_hbm, pos_hbm, neg_hbm, u128_hbm, i128_hbm,
                   out_hbm, idx_u, idx_p, idx_n, pr_u, pr_p, pr_n,
                   du0, dp0, dn0, du1, dp1, dn1, scores_v, sem0, sem1):
    wid = lax.axis_index("s") * NC + lax.axis_index("c")
    base = wid * BPW

    pltpu.sync_copy(users_hbm.at[pl.ds(base, BPW)], idx_u)
    pltpu.sync_copy(pos_hbm.at[pl.ds(base, BPW)], idx_p)
    pltpu.sync_copy(neg_hbm.at[pl.ds(base, BPW)], idx_n)

    def split(idx, pr):
        def body(g, carry):
            v = idx[pl.ds(g * L, L)]
            pr[pl.ds(g * L, L)] = v >> 2
            idx[pl.ds(g * L, L)] = v & 3
            return carry
        lax.fori_loop(0, BPW // L, body, 0)

    split(idx_u, pr_u)
    split(idx_p, pr_p)
    split(idx_n, pr_n)

    slots = [(du0, dp0, dn0, sem0), (du1, dp1, dn1, sem1)]

    def copies(c):
        du, dp, dn, sem = slots[c % 2]
        r = pl.ds(c * CH, CH)
        return (
            pltpu.make_async_copy(u128_hbm.at[pr_u.at[r]], du, sem),
            pltpu.make_async_copy(i128_hbm.at[pr_p.at[r]], dp, sem),
            pltpu.make_async_copy(i128_hbm.at[pr_n.at[r]], dn, sem),
        )

    for cp in copies(0):
        cp.start()
    for cp in copies(1):
        cp.start()

    iota = lax.iota(jnp.int32, L)
    for c in range(NCHK):
        for cp in copies(c):
            cp.wait()
        du, dp, dn, _ = slots[c % 2]

        def grp(g, carry, du=du, dp=dp, dn=dn, c=c):
            rows = g * L + iota
            o = pl.ds(c * CH + g * L, L)
            cu = idx_u[o]
            cp_ = idx_p[o]
            cn = idx_n[o]
            acc = jnp.zeros((L,), jnp.float32)
            for j in range(D):
                u = plsc.load_gather(du, [rows, cu + 4 * j])
                p = plsc.load_gather(dp, [rows, cp_ + 4 * j])
                n = plsc.load_gather(dn, [rows, cn + 4 * j])
                acc = acc + u * (n - p)
            scores_v[o] = acc
            return carry

        lax.fori_loop(0, CH // L, grp, 0)
        if c + 2 < NCHK:
            for cp in copies(c + 2):
                cp.start()

    pltpu.sync_copy(scores_v, out_hbm.at[pl.ds(base, BPW)])


_sc_scores = functools.partial(
    pl.kernel,
    mesh=plsc.VectorSubcoreMesh(core_axis_name="c", subcore_axis_name="s"),
    out_type=jax.ShapeDtypeStruct((B,), jnp.float32),
    scratch_types=(
        [pltpu.VMEM((BPW,), jnp.int32)] * 6
        + [pltpu.VMEM((CH, PADW), jnp.float32)] * 6
        + [pltpu.VMEM((BPW,), jnp.float32)]
        + [pltpu.SemaphoreType.DMA] * 2
    ),
    compiler_params=pltpu.CompilerParams(needs_layout_passes=False),
)(_scores_kernel)


def _softplus_mean_kernel(x_ref, o_ref):
    x = x_ref[...]
    o_ref[...] = (jnp.sum(jnp.log(1.0 + jnp.exp(x))) * (1.0 / B))[None, None]


def kernel(users, positive_items, negative_items, user_embedding, item_embedding):
    ut = user_embedding.T
    it = item_embedding.T
    ut_tail = jnp.pad(ut[:, NFULL * 128:], ((0, 0), (0, 128 - TAIL)))
    it_tail = jnp.pad(it[:, NFULL * 128:], ((0, 0), (0, 128 - TAIL)))
    u128, i128 = _detile(ut, it, ut_tail, it_tail)
    scores = _sc_scores(users, positive_items, negative_items, u128, i128)
    loss = pl.pallas_call(
        _softplus_mean_kernel,
        out_shape=jax.ShapeDtypeStruct((1, 1), jnp.float32),
    )(scores.reshape(128, 128))
    return loss.reshape(())
